# baseline (speedup 1.0000x reference)
import functools

import jax
import jax.numpy as jnp
from jax import lax
from jax.experimental import pallas as pl
from jax.experimental.pallas import tpu as pltpu

_SQRT1_2 = 0.7071067811865476


def _erf(x):
    a1, a2, a3, a4, a5 = (0.254829592, -0.284496736, 1.421413741,
                          -1.453152027, 1.061405429)
    p = 0.3275911
    ax = jnp.abs(x)
    t = 1.0 / (1.0 + p * ax)
    poly = ((((a5 * t + a4) * t + a3) * t + a2) * t + a1) * t
    y = 1.0 - poly * jnp.exp(-ax * ax)
    return jnp.sign(x) * y


def _gelu(x):
    return 0.5 * x * (1.0 + _erf(x * _SQRT1_2))


def _w3x3_to_mat(w):
    co = w.shape[0]
    return jnp.transpose(w, (0, 2, 3, 1)).reshape(co, 9 * w.shape[1])


def _lo16(x):
    return lax.bitcast_convert_type(x.astype(jnp.int16), jnp.bfloat16)


def _hi16(x):
    return lax.bitcast_convert_type(
        lax.shift_right_logical(x, jnp.int32(16)).astype(jnp.int16),
        jnp.bfloat16)


def _pack16(lo, hi):
    lo_u = lax.bitcast_convert_type(lo, jnp.uint16).astype(jnp.uint32)
    hi_u = lax.bitcast_convert_type(hi, jnp.uint16).astype(jnp.uint32)
    return (lo_u | (hi_u << 16)).astype(jnp.int32)


def _fused_kernel(x1_ref, x2_ref, whq_ref, wlq_ref, wx2_ref, bh1_ref,
                  bl1_ref, wha_ref, bha_ref, whb_ref, bhb_ref, wla_ref,
                  bla_ref, wlb_ref, blb_ref, whd_ref, bhd_ref, o_ref,
                  q_s, x0h_s, x0l_s, h1_s, sx_s, hi_s,
                  *, C, W2, H2, CH, TH):
    bf = jnp.bfloat16
    Lc = CH * W2
    Lp = (H2 + 2) * W2
    L1 = (TH + 2) * W2
    L2 = TH * W2

    col = lax.broadcasted_iota(jnp.int32, (1, L1), 1) % W2
    cm_m = (col != 0).astype(bf)
    cm_p = (col != W2 - 1).astype(bf)

    for ch in range(H2 // CH):
        re = x1_ref[:, pl.ds(2 * ch * CH, CH, 2), :]
        ro = x1_ref[:, pl.ds(2 * ch * CH + 1, CH, 2), :]
        q_s[0 * C:1 * C, :] = _lo16(re).reshape(C, Lc)
        q_s[1 * C:2 * C, :] = _hi16(re).reshape(C, Lc)
        q_s[2 * C:3 * C, :] = _lo16(ro).reshape(C, Lc)
        q_s[3 * C:4 * C, :] = _hi16(ro).reshape(C, Lc)
        qv = q_s[...]
        x2v = x2_ref[:, pl.ds(ch * CH, CH), :].reshape(C, Lc)
        dst = pl.ds(W2 + ch * Lc, Lc)
        x0h = jnp.dot(whq_ref[...], qv, preferred_element_type=jnp.float32)
        x0h_s[:, dst] = (x0h + bh1_ref[...]).astype(bf)
        x0l = jnp.dot(wlq_ref[...], qv, preferred_element_type=jnp.float32)
        x0l += jnp.dot(wx2_ref[...], x2v, preferred_element_type=jnp.float32)
        x0l_s[:, dst] = (x0l + bl1_ref[...]).astype(bf)
    for s in (x0h_s, x0l_s):
        s[:, pl.ds(0, W2)] = jnp.zeros((C, W2), bf)
        s[:, pl.ds((H2 + 1) * W2, W2)] = jnp.zeros((C, W2), bf)

    def conv3x3_tile(src_s, w_ref, b_ref, t):
        v = src_s[:, pl.ds(t * TH * W2, L1)]
        sm = jnp.concatenate([v[:, :1], v[:, :-1]], axis=-1) * cm_m
        sp = jnp.concatenate([v[:, 1:], v[:, -1:]], axis=-1) * cm_p
        sx_s[0:C, :] = sm
        sx_s[C:2 * C, :] = v
        sx_s[2 * C:3 * C, :] = sp
        acc = jnp.dot(w_ref[:, 0:3 * C], sx_s[:, 0:L2],
                      preferred_element_type=jnp.float32)
        acc += jnp.dot(w_ref[:, 3 * C:6 * C], sx_s[:, W2:W2 + L2],
                       preferred_element_type=jnp.float32)
        acc += jnp.dot(w_ref[:, 6 * C:9 * C], sx_s[:, 2 * W2:2 * W2 + L2],
                       preferred_element_type=jnp.float32)
        return acc + b_ref[...]

    nT = H2 // TH

    def run_branch(x0_s, w3a, b3a, w3b, b3b):
        for t in range(nT):
            h1 = _gelu(conv3x3_tile(x0_s, w3a, b3a, t))
            h1_s[:, pl.ds(W2 + t * L2, L2)] = h1.astype(bf)
        h1_s[:, pl.ds(0, W2)] = jnp.zeros((C, W2), bf)
        h1_s[:, pl.ds((H2 + 1) * W2, W2)] = jnp.zeros((C, W2), bf)
        for t in range(nT):
            h2 = _gelu(conv3x3_tile(h1_s, w3b, b3b, t))
            res = x0_s[:, pl.ds(W2 + t * L2, L2)].astype(jnp.float32)
            yield t, h2 + res

    for t, out in run_branch(x0h_s, wha_ref, bha_ref, whb_ref, bhb_ref):
        hi_s[:, pl.ds(t * L2, L2)] = out.astype(bf)

    for t, lo in run_branch(x0l_s, wla_ref, bla_ref, wlb_ref, blb_ref):
        lo2 = 0.5 * lo
        hi = hi_s[:, pl.ds(t * L2, L2)]
        y = []
        for k in range(4):
            yk = jnp.dot(whd_ref[k * C:(k + 1) * C, :], hi,
                         preferred_element_type=jnp.float32)
            y.append((yk + bhd_ref[k * C:(k + 1) * C] + lo2).astype(bf))
        we = _pack16(y[0], y[1]).reshape(C, TH, W2)
        wo = _pack16(y[2], y[3]).reshape(C, TH, W2)
        o_ref[:, pl.ds(2 * t * TH, TH, 2), :] = we
        o_ref[:, pl.ds(2 * t * TH + 1, TH, 2), :] = wo


@jax.jit
def _forward(x1, x2, params):
    B, C, H, W = x1.shape
    assert H % 2 == 0 and W % 2 == 0
    H2, W2 = H // 2, W // 2
    assert W2 % 128 == 0 and H2 % 32 == 0
    bf = jnp.bfloat16

    CH = TH = 32

    x1i = lax.bitcast_convert_type(
        x1.astype(bf).reshape(B, C, H, W2, 2), jnp.int32)
    x2l = x2
    if x2l.shape[2] != H2:
        x2l = jnp.pad(x2l, ((0, 0), (0, 0), (1, 0), (0, 0)))
    x2b = x2l.astype(bf)

    wh = params["convh1_w"]
    Wlh, Whl, Whh = wh[:, :C], wh[:, C:2 * C], wh[:, 2 * C:]
    w_h_quad = 0.5 * jnp.concatenate(
        [Wlh + Whl + Whh, Wlh - Whl - Whh, -Wlh + Whl - Whh, -Wlh - Whl + Whh],
        axis=1)
    wl = params["convl_w"]
    Wll, Wx2 = wl[:, :C], wl[:, C:]
    w_l_quad = jnp.concatenate([0.5 * Wll] * 4, axis=1)

    wh2 = params["convh2_w"]
    G1, G2, G3 = wh2[:C], wh2[C:2 * C], wh2[2 * C:]
    w_head = 0.5 * jnp.concatenate(
        [G1 + G2 + G3, G1 - G2 - G3, -G1 + G2 - G3, -G1 - G2 + G3], axis=0)
    bh = params["convh2_b"]
    g1, g2, g3 = bh[:C], bh[C:2 * C], bh[2 * C:]
    b_head = 0.5 * jnp.concatenate(
        [g1 + g2 + g3, g1 - g2 - g3, -g1 + g2 - g3, -g1 - g2 + g3], axis=0)

    wargs = [w_h_quad.astype(bf), w_l_quad.astype(bf), Wx2.astype(bf),
             params["convh1_b"].reshape(C, 1), params["convl_b"].reshape(C, 1),
             _w3x3_to_mat(params["high_w1"]).astype(bf),
             params["high_b1"].reshape(C, 1),
             _w3x3_to_mat(params["high_w2"]).astype(bf),
             params["high_b2"].reshape(C, 1),
             _w3x3_to_mat(params["low_w1"]).astype(bf),
             params["low_b1"].reshape(C, 1),
             _w3x3_to_mat(params["low_w2"]).astype(bf),
             params["low_b2"].reshape(C, 1),
             w_head.astype(bf), b_head.reshape(4 * C, 1)]

    in_specs = [
        pl.BlockSpec((None, C, H, W2), lambda b: (b, 0, 0, 0)),
        pl.BlockSpec((None, C, H2, W2), lambda b: (b, 0, 0, 0)),
    ] + [pl.BlockSpec(w.shape, lambda b: (0, 0)) for w in wargs]

    body = functools.partial(_fused_kernel, C=C, W2=W2, H2=H2, CH=CH, TH=TH)
    yi = pl.pallas_call(
        body,
        out_shape=jax.ShapeDtypeStruct((B, C, H, W2), jnp.int32),
        grid=(B,),
        in_specs=in_specs,
        out_specs=pl.BlockSpec((None, C, H, W2), lambda b: (b, 0, 0, 0)),
        scratch_shapes=[
            pltpu.VMEM((4 * C, CH * W2), bf),
            pltpu.VMEM((C, (H2 + 2) * W2), bf),
            pltpu.VMEM((C, (H2 + 2) * W2), bf),
            pltpu.VMEM((C, (H2 + 2) * W2), bf),
            pltpu.VMEM((3 * C, (TH + 2) * W2), bf),
            pltpu.VMEM((C, H2 * W2), bf),
        ],
        compiler_params=pltpu.CompilerParams(
            dimension_semantics=("parallel",),
            vmem_limit_bytes=64 << 20),
    )(x1i, x2b, *wargs)

    y = lax.bitcast_convert_type(yi, bf).reshape(B, C, H, W)
    return y.astype(x1.dtype)


def kernel(x1, x2, convh1_w, convh1_b, high_w1, high_b1, high_w2, high_b2,
           convh2_w, convh2_b, convl_w, convl_b, low_w1, low_b1, low_w2,
           low_b2):
    params = {
        "convh1_w": convh1_w, "convh1_b": convh1_b,
        "high_w1": high_w1, "high_b1": high_b1,
        "high_w2": high_w2, "high_b2": high_b2,
        "convh2_w": convh2_w, "convh2_b": convh2_b,
        "convl_w": convl_w, "convl_b": convl_b,
        "low_w1": low_w1, "low_b1": low_b1,
        "low_w2": low_w2, "low_b2": low_b2,
    }
    return _forward(x1, x2, params)
